# Initial kernel scaffold; baseline (speedup 1.0000x reference)
import functools

import jax
import jax.numpy as jnp
from jax import lax
from jax.experimental import pallas as pl
from jax.experimental.pallas import tpu as pltpu
from jax.experimental.pallas import tpu_sc as plsc

NC = 2
NS = 16
NW = NC * NS
L = 16

SHIFTS = (21, 11, 0)
MASKS = (0x3FF, 0x3FF, 0x7FF)
HSIZES = (1024, 1024, 2048)


def _srl(x, n):
  return lax.shift_right_logical(x, jnp.full(jnp.shape(x), n, jnp.int32))


def _make_sc_kernel(B, P, K):
  rows_per_w = B // NW
  mesh = plsc.VectorSubcoreMesh(core_axis_name="c", subcore_axis_name="s")

  @functools.partial(
      pl.kernel,
      out_type=jax.ShapeDtypeStruct((NW * L,), jnp.float32),
      mesh=mesh,
      scratch_types=[
          pltpu.VMEM((P,), jnp.float32),
          pltpu.VMEM((HSIZES[0],), jnp.int32),
          pltpu.VMEM((HSIZES[0],), jnp.float32),
          pltpu.VMEM((HSIZES[1],), jnp.int32),
          pltpu.VMEM((HSIZES[1],), jnp.float32),
          pltpu.VMEM((HSIZES[2],), jnp.int32),
          pltpu.VMEM((HSIZES[2],), jnp.float32),
          pltpu.VMEM((L,), jnp.float32),
      ],
  )
  def sc_kernel(loss_hbm, out_hbm, row_v, c1, s1, c2, s2, c3, s3, outv):
    cid = lax.axis_index("c")
    sid = lax.axis_index("s")
    wid = sid * NC + cid
    iota = lax.iota(jnp.int32, L)
    ones_i = jnp.ones((L,), jnp.int32)

    def zero_hist(ref, n):
      zi = jnp.zeros((L,), ref.dtype)

      def zbody(i, _):
        ref[pl.ds(i * L, L)] = zi
        return 0

      lax.fori_loop(0, n // L, zbody, 0)

    def n_chunks(m):
      if isinstance(m, int):
        return (m + L - 1) // L
      return _srl(m + (L - 1), 4)

    def hist_pass(m, shift, mbits, c_ref, s_ref):
      def hbody(i, _):
        x = row_v[pl.ds(i * L, L)]
        valid = (i * L + iota) < m
        bits = plsc.bitcast(x, jnp.int32)
        b = _srl(bits, shift) & mbits
        plsc.addupdate_scatter(c_ref, [b], ones_i, mask=valid)
        plsc.addupdate_scatter(s_ref, [b], x, mask=valid)
        return 0

      lax.fori_loop(0, n_chunks(m), hbody, 0)

    def find_bucket(c_ref, s_ref, n, k_rem):
      def fbody(i, carry):
        found, b_sel, nb, sb, cum_n, cum_s = carry
        c = c_ref[pl.ds(i * L, L)]
        s = s_ref[pl.ds(i * L, L)]
        scan_c = plsc.cumsum(c)
        scan_s = plsc.cumsum(s)
        tot_c = jnp.sum(c)
        tot_s = jnp.sum(s)
        cross = jnp.logical_and(jnp.logical_not(found),
                                (cum_n + scan_c) >= k_rem)
        lane = jnp.min(jnp.where(cross, iota, L))
        hit = lane < L
        sel = iota == lane
        c_at = jnp.sum(jnp.where(sel, c, 0))
        s_at = jnp.sum(jnp.where(sel, s, jnp.float32(0)))
        scs_at = jnp.sum(jnp.where(sel, scan_c, 0))
        sss_at = jnp.sum(jnp.where(sel, scan_s, jnp.float32(0)))
        b_sel = jnp.where(hit, i * L + lane, b_sel)
        nb = jnp.where(hit, cum_n + scs_at - c_at, nb)
        sb = jnp.where(hit, cum_s + sss_at - s_at, sb)
        found = jnp.logical_or(found, hit)
        return (found, b_sel, nb, sb, cum_n + tot_c, cum_s + tot_s)

      init = (jnp.bool_(False), jnp.int32(0), jnp.int32(0), jnp.float32(0),
              jnp.int32(0), jnp.float32(0))
      _, b_sel, nb, sb, _, _ = lax.fori_loop(0, n // L, fbody, init)
      return b_sel, nb, sb

    def compact(m, shift, mbits, b_sel):
      def cbody(i, off):
        x = row_v[pl.ds(i * L, L)]
        valid = (i * L + iota) < m
        bits = plsc.bitcast(x, jnp.int32)
        b = _srl(bits, shift) & mbits
        sel = jnp.logical_and(valid, b == b_sel)
        sel_i = sel.astype(jnp.int32)
        pm = plsc.cumsum(sel_i)
        idx = off + pm - 1
        plsc.store_scatter(row_v, [idx], x, mask=sel)
        return off + jnp.sum(sel_i)

      return lax.fori_loop(0, n_chunks(m), cbody, jnp.int32(0))

    def row_body(r, contrib_acc):
      row = wid * rows_per_w + r
      pltpu.sync_copy(loss_hbm.at[pl.ds(row * P, P)], row_v)
      zero_hist(c1, HSIZES[0])
      zero_hist(s1, HSIZES[0])
      zero_hist(c2, HSIZES[1])
      zero_hist(s2, HSIZES[1])
      zero_hist(c3, HSIZES[2])
      zero_hist(s3, HSIZES[2])

      hist_pass(P, SHIFTS[0], MASKS[0], c1, s1)
      b1, nb1, sb1 = find_bucket(c1, s1, HSIZES[0], jnp.int32(K))
      acc_n = nb1
      acc_s = sb1
      k_rem = K - nb1
      m2 = compact(P, SHIFTS[0], MASKS[0], b1)

      hist_pass(m2, SHIFTS[1], MASKS[1], c2, s2)
      b2, nb2, sb2 = find_bucket(c2, s2, HSIZES[1], k_rem)
      acc_n = acc_n + nb2
      acc_s = acc_s + sb2
      k_rem = k_rem - nb2
      m3 = compact(m2, SHIFTS[1], MASKS[1], b2)

      hist_pass(m3, SHIFTS[2], MASKS[2], c3, s3)
      b3, nb3, sb3 = find_bucket(c3, s3, HSIZES[2], k_rem)
      acc_n = acc_n + nb3
      acc_s = acc_s + sb3

      t_bits = (b1 << 21) | (b2 << 11) | b3
      t_vec = plsc.bitcast(jnp.full((L,), t_bits, jnp.int32), jnp.float32)
      contrib = acc_s + (K - acc_n).astype(jnp.float32) * t_vec
      return jnp.where(iota == r, contrib, contrib_acc)

    contrib_acc = lax.fori_loop(0, rows_per_w, row_body,
                                jnp.zeros((L,), jnp.float32))
    outv[...] = contrib_acc
    pltpu.sync_copy(outv, out_hbm.at[pl.ds(wid * L, L)])

  return sc_kernel


def _tc_mean(x_ref, o_ref, *, scale):
  o_ref[0, 0] = jnp.sum(x_ref[...]) * scale


def kernel(loss):
  B = loss.shape[0]
  P = loss.reshape(B, -1).shape[1]
  K = int(0.7 * P)
  sc_kernel = _make_sc_kernel(B, P, K)
  partials = sc_kernel(loss.reshape(-1))
  out = pl.pallas_call(
      functools.partial(_tc_mean, scale=1.0 / (B * K)),
      out_shape=jax.ShapeDtypeStruct((1, 1), jnp.float32),
  )(partials.reshape(4, NW * L // 4))
  return out[0, 0]


# speedup vs baseline: 5.8289x; 5.8289x over previous
import functools

import jax
import jax.numpy as jnp
from jax import lax
from jax.experimental import pallas as pl
from jax.experimental.pallas import tpu as pltpu
from jax.experimental.pallas import tpu_sc as plsc

NC = 2
NS = 16
NW = NC * NS
L = 16

SHIFTS = (21, 11, 0)
MASKS = (0x3FF, 0x3FF, 0x7FF)
HSIZES = (1024, 1024, 2048)


def _srl(x, n):
  return lax.shift_right_logical(x, jnp.full(jnp.shape(x), n, jnp.int32))


def _make_sc_kernel(B, P, K):
  rows_per_w = B // NW
  mesh = plsc.VectorSubcoreMesh(core_axis_name="c", subcore_axis_name="s")

  @functools.partial(
      pl.kernel,
      out_type=jax.ShapeDtypeStruct((NW * L,), jnp.float32),
      mesh=mesh,
      compiler_params=pltpu.CompilerParams(needs_layout_passes=False),
      scratch_types=[
          pltpu.VMEM((P,), jnp.float32),
          pltpu.VMEM((HSIZES[0],), jnp.int32),
          pltpu.VMEM((HSIZES[0],), jnp.float32),
          pltpu.VMEM((HSIZES[1],), jnp.int32),
          pltpu.VMEM((HSIZES[1],), jnp.float32),
          pltpu.VMEM((HSIZES[2],), jnp.int32),
          pltpu.VMEM((HSIZES[2],), jnp.float32),
          pltpu.VMEM((L,), jnp.float32),
      ],
  )
  def sc_kernel(loss_hbm, out_hbm, row_v, c1, s1, c2, s2, c3, s3, outv):
    cid = lax.axis_index("c")
    sid = lax.axis_index("s")
    wid = sid * NC + cid
    iota = lax.iota(jnp.int32, L)
    ones_i = jnp.ones((L,), jnp.int32)

    def zero_hist(ref, n):
      zi = jnp.zeros((L,), ref.dtype)

      def zbody(i, _):
        ref[pl.ds(i * L, L)] = zi
        return 0

      lax.fori_loop(0, n // L, zbody, 0)

    def n_chunks(m):
      if isinstance(m, int):
        return (m + L - 1) // L
      return _srl(m + (L - 1), 4)

    def hist_pass(m, shift, mbits, c_ref, s_ref):
      def hbody(i, _):
        x = row_v[pl.ds(i * L, L)]
        valid = (i * L + iota) < m
        bits = lax.bitcast_convert_type(x, jnp.int32)
        b = _srl(bits, shift) & mbits
        plsc.addupdate_scatter(c_ref, [b], ones_i, mask=valid)
        plsc.addupdate_scatter(s_ref, [b], x, mask=valid)
        return 0

      lax.fori_loop(0, n_chunks(m), hbody, 0)

    def find_bucket(c_ref, s_ref, n, k_rem):
      def fbody(i, carry):
        found, b_sel, nb, sb, cum_n, cum_s = carry
        c = c_ref[pl.ds(i * L, L)]
        s = s_ref[pl.ds(i * L, L)]
        scan_c = plsc.cumsum(c)
        scan_s = plsc.cumsum(s)
        tot_c = jnp.sum(c)
        tot_s = jnp.sum(s)
        cross = jnp.logical_and(jnp.logical_not(found),
                                (cum_n + scan_c) >= k_rem)
        lane = jnp.min(jnp.where(cross, iota, L))
        hit = lane < L
        sel = iota == lane
        c_at = jnp.sum(jnp.where(sel, c, 0))
        s_at = jnp.sum(jnp.where(sel, s, jnp.float32(0)))
        scs_at = jnp.sum(jnp.where(sel, scan_c, 0))
        sss_at = jnp.sum(jnp.where(sel, scan_s, jnp.float32(0)))
        b_sel = jnp.where(hit, i * L + lane, b_sel)
        nb = jnp.where(hit, cum_n + scs_at - c_at, nb)
        sb = jnp.where(hit, cum_s + sss_at - s_at, sb)
        found = jnp.logical_or(found, hit)
        return (found, b_sel, nb, sb, cum_n + tot_c, cum_s + tot_s)

      init = (jnp.bool_(False), jnp.int32(0), jnp.int32(0), jnp.float32(0),
              jnp.int32(0), jnp.float32(0))
      _, b_sel, nb, sb, _, _ = lax.fori_loop(0, n // L, fbody, init)
      return b_sel, nb, sb

    def compact(m, shift, mbits, b_sel):
      def cbody(i, off):
        x = row_v[pl.ds(i * L, L)]
        valid = (i * L + iota) < m
        bits = lax.bitcast_convert_type(x, jnp.int32)
        b = _srl(bits, shift) & mbits
        sel = jnp.logical_and(valid, b == b_sel)
        sel_i = sel.astype(jnp.int32)
        pm = plsc.cumsum(sel_i)
        idx = off + pm - 1
        plsc.store_scatter(row_v, [idx], x, mask=sel)
        return off + jnp.sum(sel_i)

      return lax.fori_loop(0, n_chunks(m), cbody, jnp.int32(0))

    def row_body(r, contrib_acc):
      row = wid * rows_per_w + r
      pltpu.sync_copy(loss_hbm.at[pl.ds(row * P, P)], row_v)
      zero_hist(c1, HSIZES[0])
      zero_hist(s1, HSIZES[0])
      zero_hist(c2, HSIZES[1])
      zero_hist(s2, HSIZES[1])
      zero_hist(c3, HSIZES[2])
      zero_hist(s3, HSIZES[2])

      hist_pass(P, SHIFTS[0], MASKS[0], c1, s1)
      b1, nb1, sb1 = find_bucket(c1, s1, HSIZES[0], jnp.int32(K))
      acc_n = nb1
      acc_s = sb1
      k_rem = K - nb1
      m2 = compact(P, SHIFTS[0], MASKS[0], b1)

      hist_pass(m2, SHIFTS[1], MASKS[1], c2, s2)
      b2, nb2, sb2 = find_bucket(c2, s2, HSIZES[1], k_rem)
      acc_n = acc_n + nb2
      acc_s = acc_s + sb2
      k_rem = k_rem - nb2
      m3 = compact(m2, SHIFTS[1], MASKS[1], b2)

      hist_pass(m3, SHIFTS[2], MASKS[2], c3, s3)
      b3, nb3, sb3 = find_bucket(c3, s3, HSIZES[2], k_rem)
      acc_n = acc_n + nb3
      acc_s = acc_s + sb3

      t_bits = (b1 << 21) | (b2 << 11) | b3
      t_vec = lax.bitcast_convert_type(jnp.full((L,), t_bits, jnp.int32), jnp.float32)
      contrib = acc_s + (K - acc_n).astype(jnp.float32) * t_vec
      return jnp.where(iota == r, contrib, contrib_acc)

    contrib_acc = lax.fori_loop(0, rows_per_w, row_body,
                                jnp.zeros((L,), jnp.float32))
    outv[...] = contrib_acc
    pltpu.sync_copy(outv, out_hbm.at[pl.ds(wid * L, L)])

  return sc_kernel


def _tc_mean(x_ref, o_ref, *, scale):
  o_ref[...] = jnp.sum(x_ref[...], keepdims=True).reshape(1, 1) * scale


def kernel(loss):
  B = loss.shape[0]
  P = loss.reshape(B, -1).shape[1]
  K = int(0.7 * P)
  sc_kernel = _make_sc_kernel(B, P, K)
  partials = sc_kernel(loss.reshape(-1))
  out = pl.pallas_call(
      functools.partial(_tc_mean, scale=1.0 / (B * K)),
      out_shape=jax.ShapeDtypeStruct((1, 1), jnp.float32),
  )(partials.reshape(4, NW * L // 4))
  return out[0, 0]


# speedup vs baseline: 7.1403x; 1.2250x over previous
import functools

import jax
import jax.numpy as jnp
from jax import lax
from jax.experimental import pallas as pl
from jax.experimental.pallas import tpu as pltpu
from jax.experimental.pallas import tpu_sc as plsc

NC = 2
NS = 16
NW = NC * NS
L = 16
U = 4

SH1, SH2, SH3 = 21, 11, 0
M1, M2, M3 = 0x3FF, 0x3FF, 0x7FF
H1, H2, H3 = 1024, 1024, 2048


def _srl(x, n):
  return lax.shift_right_logical(x, jnp.full(jnp.shape(x), n, jnp.int32))


def _bits(x):
  return lax.bitcast_convert_type(x, jnp.int32)


def _make_sc_kernel(B, P, K):
  rows_per_w = B // NW
  mesh = plsc.VectorSubcoreMesh(core_axis_name="c", subcore_axis_name="s")

  @functools.partial(
      pl.kernel,
      out_type=jax.ShapeDtypeStruct((NW * L,), jnp.float32),
      mesh=mesh,
      compiler_params=pltpu.CompilerParams(needs_layout_passes=False),
      scratch_types=[
          pltpu.VMEM((P,), jnp.float32),
          pltpu.VMEM((P,), jnp.float32),
          pltpu.VMEM((H1,), jnp.int32),
          pltpu.VMEM((H2,), jnp.int32),
          pltpu.VMEM((H3,), jnp.int32),
          pltpu.VMEM((L,), jnp.float32),
      ],
  )
  def sc_kernel(loss_hbm, out_hbm, row_v, buf_v, c1, c2, c3, outv):
    cid = lax.axis_index("c")
    sid = lax.axis_index("s")
    wid = sid * NC + cid
    iota = lax.iota(jnp.int32, L)
    ones_i = jnp.ones((L,), jnp.int32)
    zeros_i = jnp.zeros((L,), jnp.int32)
    zeros_f = jnp.zeros((L,), jnp.float32)

    def zero_ref(ref, n):
      def zbody(i, _):
        ref[pl.ds(i * L, L)] = zeros_i
        return 0

      lax.fori_loop(0, n // L, zbody, 0)

    zero_ref(c1, H1)
    zero_ref(c2, H2)
    zero_ref(c3, H3)

    def find_count(c_ref, nchunks, k_rem):
      def fbody(i, carry):
        cum, found, cum_sel, base_sel, c_sel = carry
        c = c_ref[pl.ds(i * L, L)]
        tot = jnp.sum(c)
        c_ref[pl.ds(i * L, L)] = zeros_i
        hit = jnp.logical_and(jnp.logical_not(found), (cum + tot) >= k_rem)
        cum_sel = jnp.where(hit, cum, cum_sel)
        base_sel = jnp.where(hit, i * L, base_sel)
        c_sel = jnp.where(hit, c, c_sel)
        return (cum + tot, jnp.logical_or(found, hit), cum_sel, base_sel,
                c_sel)

      init = (jnp.int32(0), jnp.bool_(False), jnp.int32(0), jnp.int32(0),
              zeros_i)
      _, _, cum_sel, base_sel, c_sel = lax.fori_loop(0, nchunks, fbody, init)
      scan_c = plsc.cumsum(c_sel)
      cross = (cum_sel + scan_c) >= k_rem
      lane = jnp.min(jnp.where(cross, iota, L - 1))
      nb = cum_sel + jnp.sum(jnp.where(iota < lane, c_sel, 0))
      return base_sel + lane, nb

    def hist1():
      def hbody(i, _):
        for u in range(U):
          x = row_v[pl.ds((i * U + u) * L, L)]
          b = _srl(_bits(x), SH1) & M1
          plsc.addupdate_scatter(c1, [b], ones_i)
        return 0

      lax.fori_loop(0, P // (L * U), hbody, 0)

    def compact(src, dst, m, shift, lvl_mask, b_sel, prefix_lo, nshift,
                nmask, c_next):
      def cbody(i, carry):
        off, sacc = carry
        for u in range(U):
          base = (i * U + u) * L
          x = src[pl.ds(base, L)]
          bits = _bits(x)
          valid = (base + iota) < m
          below = jnp.logical_and(valid, bits < prefix_lo)
          sacc = sacc + jnp.where(below, x, jnp.float32(0))
          sel = jnp.logical_and(valid, (_srl(bits, shift) & lvl_mask) == b_sel)
          sel_i = sel.astype(jnp.int32)
          pm = plsc.cumsum(sel_i)
          plsc.store_scatter(dst, [off + pm - 1], x, mask=sel)
          bn = _srl(bits, nshift) & nmask
          plsc.addupdate_scatter(c_next, [bn], ones_i, mask=sel)
          off = off + jnp.sum(sel_i)
        return (off, sacc)

      nch = _srl(m + (L * U - 1), 6) if not isinstance(m, int) else (
          (m + L * U - 1) // (L * U))
      return lax.fori_loop(0, nch, cbody, (jnp.int32(0), zeros_f))

    def below_sum(src, m, t_bits):
      def mbody(i, sacc):
        base = i * L
        x = src[pl.ds(base, L)]
        below = jnp.logical_and((base + iota) < m, _bits(x) < t_bits)
        return sacc + jnp.where(below, x, jnp.float32(0))

      return lax.fori_loop(0, _srl(m + (L - 1), 4), mbody, zeros_f)

    def row_body(r, contrib_acc):
      row = wid * rows_per_w + r
      pltpu.sync_copy(loss_hbm.at[pl.ds(row * P, P)], row_v)

      hist1()
      b1, nb1 = find_count(c1, H1 // L, jnp.int32(K))
      p1 = b1 << SH1
      m2, sacc1 = compact(row_v, buf_v, P, SH1, M1, b1, p1, SH2, M2, c2)

      b2, nb2 = find_count(c2, H2 // L, K - nb1)
      p2 = p1 | (b2 << SH2)
      m3, sacc2 = compact(buf_v, row_v, m2, SH2, M2, b2, p2, SH3, M3, c3)

      b3, nb3 = find_count(c3, H3 // L, K - nb1 - nb2)
      t_bits = p2 | b3
      sacc3 = below_sum(row_v, m3, t_bits)

      acc_n = nb1 + nb2 + nb3
      t_vec = lax.bitcast_convert_type(jnp.full((L,), t_bits, jnp.int32),
                                       jnp.float32)
      contrib = (sacc1 + sacc2 + sacc3
                 + (K - acc_n).astype(jnp.float32) * t_vec * (1.0 / L))
      csum = jnp.sum(contrib)
      return jnp.where(iota == r, csum, contrib_acc)

    contrib_acc = lax.fori_loop(0, rows_per_w, row_body, zeros_f)
    outv[...] = contrib_acc
    pltpu.sync_copy(outv, out_hbm.at[pl.ds(wid * L, L)])

  return sc_kernel


def _tc_mean(x_ref, o_ref, *, scale):
  o_ref[...] = jnp.sum(x_ref[...], keepdims=True).reshape(1, 1) * scale


def kernel(loss):
  B = loss.shape[0]
  P = loss.reshape(B, -1).shape[1]
  K = int(0.7 * P)
  sc_kernel = _make_sc_kernel(B, P, K)
  partials = sc_kernel(loss.reshape(-1))
  out = pl.pallas_call(
      functools.partial(_tc_mean, scale=1.0 / (B * K)),
      out_shape=jax.ShapeDtypeStruct((1, 1), jnp.float32),
  )(partials.reshape(4, NW * L // 4))
  return out[0, 0]
